# baseline (speedup 1.0000x reference)
import functools

import jax
import jax.numpy as jnp
from jax import lax
from jax.experimental import pallas as pl
from jax.experimental.pallas import tpu as pltpu
from jax.experimental.pallas import tpu_sc as plsc

B = 4096
D = 64
K = 5
NC = 2
NS = 16
L = 16
NW = NC * NS
BW = B // NW
R = B * (K + 2)


def _sc_gather(uid, pid, nid_flat, user_embed, book_embed):
    mesh = plsc.VectorSubcoreMesh(
        core_axis_name="c", subcore_axis_name="s", num_cores=NC, num_subcores=NS
    )

    @functools.partial(
        pl.kernel,
        out_type=jax.ShapeDtypeStruct((R, D), jnp.float32),
        mesh=mesh,
        scratch_types=[
            pltpu.VMEM((BW,), jnp.int32),
            pltpu.VMEM((BW,), jnp.int32),
            pltpu.VMEM((K * BW,), jnp.int32),
            pltpu.VMEM((2, BW, D), jnp.float32),
            pltpu.SemaphoreType.DMA,
            pltpu.SemaphoreType.DMA,
            pltpu.SemaphoreType.DMA,
        ],
        compiler_params=pltpu.CompilerParams(use_tc_tiling_on_sc=True),
    )
    def body(uid_hbm, pid_hbm, nid_hbm, uemb_hbm, bemb_hbm, out_hbm,
             uid_v, pid_v, nid_v, bufs, gsem, osem0, osem1):
        osems = (osem0, osem1)
        wid = lax.axis_index("s") * NC + lax.axis_index("c")
        base = wid * BW

        pltpu.sync_copy(uid_hbm.at[pl.ds(base, BW)], uid_v)
        pltpu.sync_copy(pid_hbm.at[pl.ds(base, BW)], pid_v)
        pltpu.sync_copy(nid_hbm.at[pl.ds(wid * (K * BW), K * BW)], nid_v)

        chunks = [(uid_v, 0, uemb_hbm, base), (pid_v, 0, bemb_hbm, B + base)]
        for kk in range(K):
            chunks.append((nid_v, kk * BW, bemb_hbm,
                           2 * B + wid * (K * BW) + kk * BW))

        out_cps = [None, None]
        for c, (idref, idoff, tbl, obase) in enumerate(chunks):
            buf = bufs.at[c % 2]
            if out_cps[c % 2] is not None:
                out_cps[c % 2].wait()

            def fire(ci, carry, idref=idref, idoff=idoff, tbl=tbl, buf=buf):
                idvec = idref[pl.ds(idoff + ci * L, L)]
                for j in range(L):
                    rid = idvec[j]
                    pltpu.async_copy(tbl.at[pl.ds(rid, 1), :],
                                     buf.at[pl.ds(ci * L + j, 1), :], gsem)
                return carry

            lax.fori_loop(0, BW // L, fire, 0)
            pltpu.make_async_copy(uemb_hbm.at[pl.ds(0, BW), :], buf, gsem).wait()
            out_cps[c % 2] = pltpu.async_copy(
                buf, out_hbm.at[pl.ds(obase, BW), :], osems[c % 2])
        out_cps[0].wait()
        out_cps[1].wait()

    return body(uid, pid, nid_flat, user_embed, book_embed)


def _tc_loss(rows):
    def tc_body(rows_ref, o_ref):
        u = rows_ref[0:B, :]
        p = rows_ref[B:2 * B, :]
        n = rows_ref[2 * B:R, :]
        pos_score = jnp.sum(u * p, axis=1)
        n3 = n.reshape(B, K, D)
        u3 = u.reshape(B, 1, D)
        neg_score = jnp.sum(n3 * u3, axis=2)
        pos_term = jnp.log(1.0 / (1.0 + jnp.exp(-pos_score)) + 1e-10)
        neg_term = jnp.log(1.0 / (1.0 + jnp.exp(neg_score)) + 1e-10)
        total = jnp.sum(pos_term) + jnp.sum(neg_term)
        o_ref[0, 0] = -total / jnp.float32(B)

    return pl.pallas_call(
        tc_body,
        out_shape=jax.ShapeDtypeStruct((1, 1), jnp.float32),
        out_specs=pl.BlockSpec(memory_space=pltpu.SMEM),
    )(rows)


def kernel(user_ids, pos_book_ids, neg_book_ids, user_embed, book_embed):
    uid = user_ids.astype(jnp.int32)
    pid = pos_book_ids.astype(jnp.int32)
    nid_flat = neg_book_ids.astype(jnp.int32).reshape(K * B)
    rows = _sc_gather(uid, pid, nid_flat, user_embed, book_embed)
    loss = _tc_loss(rows)
    return loss.reshape(())


# speedup vs baseline: 1.1722x; 1.1722x over previous
import functools

import jax
import jax.numpy as jnp
from jax import lax
from jax.experimental import pallas as pl
from jax.experimental.pallas import tpu as pltpu
from jax.experimental.pallas import tpu_sc as plsc

B = 4096
D = 64
K = 5
NC = 2
NS = 16
L = 16
NW = NC * NS
BW = B // NW
R = B * (K + 2)
RP = R // 2


def _sc_gather(uid, pid, nid_flat, user_embed, book_embed):
    mesh = plsc.VectorSubcoreMesh(
        core_axis_name="c", subcore_axis_name="s", num_cores=NC, num_subcores=NS
    )

    @functools.partial(
        pl.kernel,
        out_type=jax.ShapeDtypeStruct((R, D), jnp.float32),
        mesh=mesh,
        scratch_types=[
            pltpu.VMEM((BW,), jnp.int32),
            pltpu.VMEM((BW,), jnp.int32),
            pltpu.VMEM((K * BW,), jnp.int32),
            pltpu.VMEM((2, BW, D), jnp.float32),
            pltpu.SemaphoreType.DMA,
            pltpu.SemaphoreType.DMA,
            pltpu.SemaphoreType.DMA,
        ],
        compiler_params=pltpu.CompilerParams(use_tc_tiling_on_sc=True),
    )
    def body(uid_hbm, pid_hbm, nid_hbm, uemb_hbm, bemb_hbm, out_hbm,
             uid_v, pid_v, nid_v, bufs, gsem, osem0, osem1):
        osems = (osem0, osem1)
        wid = lax.axis_index("s") * NC + lax.axis_index("c")
        base = wid * BW

        pltpu.sync_copy(uid_hbm.at[pl.ds(base, BW)], uid_v)
        pltpu.sync_copy(pid_hbm.at[pl.ds(base, BW)], pid_v)
        for kk in range(K):
            pltpu.sync_copy(nid_hbm.at[pl.ds(kk * B + base, BW)],
                            nid_v.at[pl.ds(kk * BW, BW)])

        chunks = [(uid_v, 0, uemb_hbm, base), (pid_v, 0, bemb_hbm, B + base)]
        for kk in range(K):
            chunks.append((nid_v, kk * BW, bemb_hbm, (2 + kk) * B + base))

        out_cps = [None, None]
        for c, (idref, idoff, tbl, obase) in enumerate(chunks):
            buf = bufs.at[c % 2]
            if out_cps[c % 2] is not None:
                out_cps[c % 2].wait()

            def fire(ci, carry, idref=idref, idoff=idoff, tbl=tbl, buf=buf):
                idvec = idref[pl.ds(idoff + ci * L, L)]
                for j in range(L):
                    rid = idvec[j]
                    pltpu.async_copy(tbl.at[pl.ds(rid, 1), :],
                                     buf.at[pl.ds(ci * L + j, 1), :], gsem)
                return carry

            lax.fori_loop(0, BW // L, fire, 0)
            pltpu.make_async_copy(
                uemb_hbm.at[pl.ds(0, BW), :], buf, gsem).wait()
            out_cps[c % 2] = pltpu.async_copy(
                buf,
                out_hbm.at[pl.ds(pl.multiple_of(obase, 128), BW), :],
                osems[c % 2])
        out_cps[0].wait()
        out_cps[1].wait()

    return body(uid, pid, nid_flat, user_embed, book_embed)


def _tc_loss(rows):
    def tc_body(rows_ref, o_ref):
        u = rows_ref[0:B, :]
        rest = rows_ref[B:(2 + K) * B, :]
        ut = jnp.concatenate([u] * (K + 1), axis=0)
        q = rest * ut
        ones_m = jnp.ones((D, 128), jnp.float32)
        s = jax.lax.dot_general(
            q, ones_m, (((1,), (0,)), ((), ())),
            preferred_element_type=jnp.float32)
        rid = lax.broadcasted_iota(jnp.int32, ((K + 1) * B, 128), 0)
        s = jnp.where(rid < B, s, -s)
        t = jnp.log(1.0 / (1.0 + jnp.exp(-s)) + 1e-10)
        o_ref[0, 0] = -jnp.sum(t) / jnp.float32(128 * B)

    return pl.pallas_call(
        tc_body,
        out_shape=jax.ShapeDtypeStruct((1, 1), jnp.float32),
        out_specs=pl.BlockSpec(memory_space=pltpu.SMEM),
    )(rows)


def kernel(user_ids, pos_book_ids, neg_book_ids, user_embed, book_embed):
    uid = user_ids.astype(jnp.int32)
    pid = pos_book_ids.astype(jnp.int32)
    nid_flat = neg_book_ids.astype(jnp.int32).T.reshape(K * B)
    rows = _sc_gather(uid, pid, nid_flat, user_embed, book_embed)
    loss = _tc_loss(rows)
    return loss.reshape(())
